# Initial kernel scaffold; baseline (speedup 1.0000x reference)
import jax
import jax.numpy as jnp
from jax.experimental import pallas as pl


def kernel(x, edge_index, W_self, b_self, W_neigh, b_neigh):
    raise NotImplementedError("write your pallas kernel here")


# speedup vs baseline: 2.6941x; 2.6941x over previous
import functools

import jax
import jax.numpy as jnp
from jax import lax
from jax.experimental import pallas as pl
from jax.experimental.pallas import tpu as pltpu
from jax.experimental.pallas import tpu_sc as plsc

NC = 2
NS = 16
K = 128
H = 128
G = 8


def _zero_chunk_starts(rpt):
    starts = [t * K for t in range((rpt + K - 1) // K)]
    starts[-1] = rpt - K
    return starts


def _sc_aggregate(x2, gi3d, dst2d, zrow, ones, iden, n_pad, n_chunks):
    rpt = n_pad // NS
    cpt = n_chunks // NS
    ngrp = cpt // G
    starts = _zero_chunk_starts(rpt)
    mesh = plsc.VectorSubcoreMesh(core_axis_name="core", subcore_axis_name="subcore")

    @functools.partial(
        pl.kernel,
        out_type=[
            jax.ShapeDtypeStruct((NC, n_pad, H), jnp.float32),
            jax.ShapeDtypeStruct((NC, n_pad, H), jnp.float32),
        ],
        mesh=mesh,
        scratch_types=[
            pltpu.VMEM((G, K), jnp.int32),
            pltpu.VMEM((G, K), jnp.int32),
            pltpu.VMEM((K, H), jnp.float32),
            pltpu.VMEM((1, K), jnp.int32),
            pltpu.VMEM_SHARED((n_pad, H), jnp.float32),
            pltpu.SemaphoreType.DMA,
        ],
    )
    def k(x2_hbm, gi_hbm, dst_hbm, zrow_hbm, ones_hbm, iden_hbm,
          agg_hbm, deg_hbm,
          sidx, didx, gbuf, zidx, acc_sh, sem):
        c = lax.axis_index("core")
        s = lax.axis_index("subcore")
        row0 = s * cpt

        def zero_acc():
            pltpu.sync_copy(zrow_hbm, gbuf)
            for ti in range(len(starts)):
                pltpu.sync_copy(iden_hbm.at[s, ti], zidx)
                pltpu.sync_copy(gbuf, acc_sh.at[zidx.at[0]])

        def write_acc(out_hbm):
            for ti, st in enumerate(starts):
                ro = pl.multiple_of(s * rpt + st, 8)
                pltpu.sync_copy(iden_hbm.at[s, ti], zidx)
                pltpu.async_copy(acc_sh.at[zidx.at[0]], gbuf, sem).wait()
                pltpu.sync_copy(gbuf, out_hbm.at[c, pl.ds(ro, K)])

        zero_acc()
        plsc.subcore_barrier()

        @pl.loop(0, ngrp)
        def _(g):
            base = pl.multiple_of(row0 + g * G, 8)
            pltpu.sync_copy(gi_hbm.at[c, pl.ds(base, G)], sidx)
            pltpu.sync_copy(dst_hbm.at[pl.ds(base, G)], didx)

            @pl.loop(0, G)
            def _(r):
                pltpu.async_copy(x2_hbm.at[sidx.at[r]], gbuf, sem).wait()
                pltpu.sync_copy(gbuf, acc_sh.at[didx.at[r]], add=True)

        plsc.subcore_barrier()
        write_acc(agg_hbm)
        plsc.subcore_barrier()

        zero_acc()
        plsc.subcore_barrier()
        pltpu.sync_copy(ones_hbm, gbuf)

        @pl.loop(0, ngrp)
        def _(g):
            base = pl.multiple_of(row0 + g * G, 8)
            pltpu.sync_copy(dst_hbm.at[pl.ds(base, G)], didx)

            @pl.loop(0, G)
            def _(r):
                pltpu.sync_copy(gbuf, acc_sh.at[didx.at[r]], add=True)

        plsc.subcore_barrier()
        write_acc(deg_hbm)

    return k(x2, gi3d, dst2d, zrow, ones, iden)


def _tc_body(x_ref, a0_ref, a1_ref, d0_ref, d1_ref,
             wst_ref, wn0_ref, wn1_ref, b_ref, o_ref):
    deg = (d0_ref[:, 0:1] + d1_ref[:, 0:1]) * 0.5
    recip = 1.0 / jnp.maximum(deg, 1.0)
    dot = functools.partial(jnp.dot, precision=jax.lax.Precision.HIGHEST,
                            preferred_element_type=jnp.float32)
    neigh = dot(a0_ref[...], wn0_ref[...]) + dot(a1_ref[...], wn1_ref[...])
    o_ref[...] = dot(x_ref[...], wst_ref[...]) + neigh * recip + b_ref[...]


def kernel(x, edge_index, W_self, b_self, W_neigh, b_neigh):
    n, d = x.shape
    e = edge_index.shape[1]
    epg = NS * K * G
    e_pad = ((e + epg - 1) // epg) * epg
    n_chunks = e_pad // K
    n_pad = ((n + 1 + 127) // 128) * 128
    rpt = n_pad // NS

    x2 = x.reshape(n * 2, H)
    pad = e_pad - e
    src_p = jnp.concatenate([edge_index[0], jnp.zeros((pad,), jnp.int32)])
    dst_p = jnp.concatenate([edge_index[1], jnp.full((pad,), n, jnp.int32)])
    gi3d = (2 * src_p + jnp.array([[0], [1]], jnp.int32)).reshape(NC, n_chunks, K)
    dst2d = dst_p.reshape(n_chunks, K)
    zrow = jnp.zeros((K, H), jnp.float32)
    ones = jnp.ones((K, H), jnp.float32)
    starts = jnp.array(_zero_chunk_starts(rpt), jnp.int32)
    iden = (jnp.arange(NS, dtype=jnp.int32)[:, None, None] * rpt
            + starts[None, :, None]
            + jnp.arange(K, dtype=jnp.int32)[None, None, :]
            ).reshape(NS, len(_zero_chunk_starts(rpt)), 1, K)

    agg, degp = _sc_aggregate(x2, gi3d, dst2d, zrow, ones, iden,
                              n_pad, n_chunks)

    wst = W_self.T
    wn0 = W_neigh[:, :H].T
    wn1 = W_neigh[:, H:].T
    bias = (b_self + b_neigh).reshape(1, d)

    bn = 256
    nb = (n + bn - 1) // bn
    out = pl.pallas_call(
        _tc_body,
        grid=(nb,),
        in_specs=[
            pl.BlockSpec((bn, d), lambda i: (i, 0)),
            pl.BlockSpec((bn, H), lambda i: (i, 0)),
            pl.BlockSpec((bn, H), lambda i: (i, 0)),
            pl.BlockSpec((bn, H), lambda i: (i, 0)),
            pl.BlockSpec((bn, H), lambda i: (i, 0)),
            pl.BlockSpec((d, d), lambda i: (0, 0)),
            pl.BlockSpec((H, d), lambda i: (0, 0)),
            pl.BlockSpec((H, d), lambda i: (0, 0)),
            pl.BlockSpec((1, d), lambda i: (0, 0)),
        ],
        out_specs=pl.BlockSpec((bn, d), lambda i: (i, 0)),
        out_shape=jax.ShapeDtypeStruct((n, d), jnp.float32),
    )(x, agg[0], agg[1], degp[0], degp[1], wst, wn0, wn1, bias)
    return out


# speedup vs baseline: 3.1917x; 1.1847x over previous
import functools

import jax
import jax.numpy as jnp
from jax import lax
from jax.experimental import pallas as pl
from jax.experimental.pallas import tpu as pltpu
from jax.experimental.pallas import tpu_sc as plsc

NC = 2
NS = 16
K = 128
H = 128
G = 8


def _zero_chunk_starts(rpt):
    starts = [t * K for t in range((rpt + K - 1) // K)]
    starts[-1] = rpt - K
    return starts


def _sc_aggregate(x2, gi3d, dst2d, zrow, ones, iden, n_pad, n_chunks):
    rpt = n_pad // NS
    cpt = n_chunks // NS
    ngrp = cpt // G
    starts = _zero_chunk_starts(rpt)
    mesh = plsc.VectorSubcoreMesh(core_axis_name="core", subcore_axis_name="subcore")

    @functools.partial(
        pl.kernel,
        out_type=[
            jax.ShapeDtypeStruct((NC, n_pad, H), jnp.float32),
            jax.ShapeDtypeStruct((NC, n_pad, H), jnp.float32),
        ],
        mesh=mesh,
        scratch_types=[
            pltpu.VMEM((G, K), jnp.int32),
            pltpu.VMEM((G, K), jnp.int32),
            pltpu.VMEM((K, H), jnp.float32),
            pltpu.VMEM((K, H), jnp.float32),
            pltpu.VMEM((1, K), jnp.int32),
            pltpu.VMEM_SHARED((n_pad, H), jnp.float32),
            pltpu.SemaphoreType.DMA,
            pltpu.SemaphoreType.DMA,
            pltpu.SemaphoreType.DMA,
        ],
    )
    def k(x2_hbm, gi_hbm, dst_hbm, zrow_hbm, ones_hbm, iden_hbm,
          agg_hbm, deg_hbm,
          sidx, didx, gbufA, gbufB, zidx, acc_sh, gsem, ssemA, ssemB):
        c = lax.axis_index("core")
        s = lax.axis_index("subcore")
        row0 = s * cpt

        def zero_acc():
            pltpu.sync_copy(zrow_hbm, gbufA)
            for ti in range(len(starts)):
                pltpu.sync_copy(iden_hbm.at[s, ti], zidx)
                pltpu.sync_copy(gbufA, acc_sh.at[zidx.at[0]])

        def write_acc(out_hbm):
            for ti, st in enumerate(starts):
                ro = pl.multiple_of(s * rpt + st, 8)
                pltpu.sync_copy(iden_hbm.at[s, ti], zidx)
                pltpu.async_copy(acc_sh.at[zidx.at[0]], gbufA, gsem).wait()
                pltpu.sync_copy(gbufA, out_hbm.at[c, pl.ds(ro, K)])

        zero_acc()
        plsc.subcore_barrier()

        bufs = (gbufA, gbufB)
        ssems = (ssemA, ssemB)

        @pl.loop(0, ngrp)
        def _(g):
            base = pl.multiple_of(row0 + g * G, 8)
            pltpu.sync_copy(gi_hbm.at[c, pl.ds(base, G)], sidx)
            pltpu.sync_copy(dst_hbm.at[pl.ds(base, G)], didx)
            sdesc = [None, None]
            for r in range(G):
                b = r & 1
                if sdesc[b] is not None:
                    sdesc[b].wait()
                pltpu.async_copy(x2_hbm.at[sidx.at[r]], bufs[b], gsem).wait()
                sdesc[b] = pltpu.async_copy(bufs[b], acc_sh.at[didx.at[r]],
                                            ssems[b], add=True)
            sdesc[0].wait()
            sdesc[1].wait()

        plsc.subcore_barrier()
        write_acc(agg_hbm)
        plsc.subcore_barrier()

        zero_acc()
        plsc.subcore_barrier()
        pltpu.sync_copy(ones_hbm, gbufA)

        @pl.loop(0, ngrp // 2)
        def _(g):
            base2 = pl.multiple_of(
                c * (n_chunks // 2) + s * (cpt // 2) + g * G, 8)
            pltpu.sync_copy(dst_hbm.at[pl.ds(base2, G)], didx)
            descs = [pltpu.async_copy(gbufA, acc_sh.at[didx.at[r]],
                                      ssemA, add=True) for r in range(G)]
            for d_ in descs:
                d_.wait()

        plsc.subcore_barrier()
        write_acc(deg_hbm)

    return k(x2, gi3d, dst2d, zrow, ones, iden)


def _tc_body(x_ref, a0_ref, a1_ref, d0_ref, d1_ref,
             wst_ref, wn0_ref, wn1_ref, b_ref, o_ref):
    deg = d0_ref[:, 0:1] + d1_ref[:, 0:1]
    recip = 1.0 / jnp.maximum(deg, 1.0)
    dot = functools.partial(jnp.dot, precision=jax.lax.Precision.HIGHEST,
                            preferred_element_type=jnp.float32)
    neigh = dot(a0_ref[...], wn0_ref[...]) + dot(a1_ref[...], wn1_ref[...])
    o_ref[...] = dot(x_ref[...], wst_ref[...]) + neigh * recip + b_ref[...]


def kernel(x, edge_index, W_self, b_self, W_neigh, b_neigh):
    n, d = x.shape
    e = edge_index.shape[1]
    epg = NS * K * G * 2
    e_pad = ((e + epg - 1) // epg) * epg
    n_chunks = e_pad // K
    n_pad = ((n + 1 + 127) // 128) * 128
    rpt = n_pad // NS

    x2 = x.reshape(n * 2, H)
    pad = e_pad - e
    src_p = jnp.concatenate([edge_index[0], jnp.zeros((pad,), jnp.int32)])
    dst_p = jnp.concatenate([edge_index[1], jnp.full((pad,), n, jnp.int32)])
    gi3d = (2 * src_p + jnp.array([[0], [1]], jnp.int32)).reshape(NC, n_chunks, K)
    dst2d = dst_p.reshape(n_chunks, K)
    zrow = jnp.zeros((K, H), jnp.float32)
    ones = jnp.ones((K, H), jnp.float32)
    starts = jnp.array(_zero_chunk_starts(rpt), jnp.int32)
    iden = (jnp.arange(NS, dtype=jnp.int32)[:, None, None] * rpt
            + starts[None, :, None]
            + jnp.arange(K, dtype=jnp.int32)[None, None, :]
            ).reshape(NS, len(_zero_chunk_starts(rpt)), 1, K)

    agg, degp = _sc_aggregate(x2, gi3d, dst2d, zrow, ones, iden,
                              n_pad, n_chunks)

    wst = W_self.T
    wn0 = W_neigh[:, :H].T
    wn1 = W_neigh[:, H:].T
    bias = (b_self + b_neigh).reshape(1, d)

    bn = 256
    nb = (n + bn - 1) // bn
    out = pl.pallas_call(
        _tc_body,
        grid=(nb,),
        in_specs=[
            pl.BlockSpec((bn, d), lambda i: (i, 0)),
            pl.BlockSpec((bn, H), lambda i: (i, 0)),
            pl.BlockSpec((bn, H), lambda i: (i, 0)),
            pl.BlockSpec((bn, H), lambda i: (i, 0)),
            pl.BlockSpec((bn, H), lambda i: (i, 0)),
            pl.BlockSpec((d, d), lambda i: (0, 0)),
            pl.BlockSpec((H, d), lambda i: (0, 0)),
            pl.BlockSpec((H, d), lambda i: (0, 0)),
            pl.BlockSpec((1, d), lambda i: (0, 0)),
        ],
        out_specs=pl.BlockSpec((bn, d), lambda i: (i, 0)),
        out_shape=jax.ShapeDtypeStruct((n, d), jnp.float32),
    )(x, agg[0], agg[1], degp[0], degp[1], wst, wn0, wn1, bias)
    return out


# speedup vs baseline: 3.3332x; 1.0443x over previous
import functools

import jax
import jax.numpy as jnp
from jax import lax
from jax.experimental import pallas as pl
from jax.experimental.pallas import tpu as pltpu
from jax.experimental.pallas import tpu_sc as plsc

NC = 2
NS = 16
K = 128
H = 128
G = 8


def _zero_chunk_starts(rpt):
    starts = [t * K for t in range((rpt + K - 1) // K)]
    starts[-1] = rpt - K
    return starts


def _sc_aggregate(x2, gi3d, dst2d, zrow, ones, iden, n_pad, n_chunks):
    rpt = n_pad // NS
    cpt = n_chunks // NS
    ngrp = cpt // G
    starts = _zero_chunk_starts(rpt)
    mesh = plsc.VectorSubcoreMesh(core_axis_name="core", subcore_axis_name="subcore")

    @functools.partial(
        pl.kernel,
        out_type=[
            jax.ShapeDtypeStruct((NC, n_pad, H), jnp.float32),
            jax.ShapeDtypeStruct((NC, n_pad, H), jnp.float32),
        ],
        mesh=mesh,
        scratch_types=[
            pltpu.VMEM((G, K), jnp.int32),
            pltpu.VMEM((G, K), jnp.int32),
            pltpu.VMEM((K, H), jnp.float32),
            pltpu.VMEM((K, H), jnp.float32),
            pltpu.VMEM((1, K), jnp.int32),
            pltpu.VMEM_SHARED((n_pad, H), jnp.float32),
            pltpu.SemaphoreType.DMA,
            pltpu.SemaphoreType.DMA,
            pltpu.SemaphoreType.DMA,
        ],
    )
    def k(x2_hbm, gi_hbm, dst_hbm, zrow_hbm, ones_hbm, iden_hbm,
          agg_hbm, deg_hbm,
          sidx, didx, gbufA, gbufB, zidx, acc_sh, gsem, ssemA, ssemB):
        c = lax.axis_index("core")
        s = lax.axis_index("subcore")
        row0 = s * cpt

        def zero_acc():
            pltpu.sync_copy(zrow_hbm, gbufA)
            for ti in range(len(starts)):
                pltpu.sync_copy(iden_hbm.at[s, ti], zidx)
                pltpu.sync_copy(gbufA, acc_sh.at[zidx.at[0]])

        def write_acc(out_hbm):
            for ti, st in enumerate(starts):
                ro = pl.multiple_of(s * rpt + st, 8)
                pltpu.sync_copy(iden_hbm.at[s, ti], zidx)
                pltpu.async_copy(acc_sh.at[zidx.at[0]], gbufA, gsem).wait()
                pltpu.sync_copy(gbufA, out_hbm.at[c, pl.ds(ro, K)])

        zero_acc()
        plsc.subcore_barrier()

        bufs = (gbufA, gbufB)
        ssems = (ssemA, ssemB)

        @pl.loop(0, ngrp)
        def _(g):
            base = pl.multiple_of(row0 + g * G, 8)
            pltpu.sync_copy(gi_hbm.at[c, pl.ds(base, G)], sidx)
            pltpu.sync_copy(dst_hbm.at[pl.ds(base, G)], didx)
            sdesc = [None, None]
            gdesc = [None, None]
            gdesc[0] = pltpu.async_copy(x2_hbm.at[sidx.at[0]], bufs[0], gsem)
            for r in range(G):
                b = r & 1
                nb = 1 - b
                if r + 1 < G:
                    if sdesc[nb] is not None:
                        sdesc[nb].wait()
                    gdesc[nb] = pltpu.async_copy(x2_hbm.at[sidx.at[r + 1]],
                                                 bufs[nb], gsem)
                gdesc[b].wait()
                sdesc[b] = pltpu.async_copy(bufs[b], acc_sh.at[didx.at[r]],
                                            ssems[b], add=True)
            sdesc[0].wait()
            sdesc[1].wait()

        plsc.subcore_barrier()
        write_acc(agg_hbm)
        plsc.subcore_barrier()

        zero_acc()
        plsc.subcore_barrier()
        pltpu.sync_copy(ones_hbm, gbufA)

        @pl.loop(0, ngrp // 2)
        def _(g):
            base2 = pl.multiple_of(
                c * (n_chunks // 2) + s * (cpt // 2) + g * G, 8)
            pltpu.sync_copy(dst_hbm.at[pl.ds(base2, G)], didx)
            descs = [pltpu.async_copy(gbufA, acc_sh.at[didx.at[r]],
                                      ssemA, add=True) for r in range(G)]
            for d_ in descs:
                d_.wait()

        plsc.subcore_barrier()
        write_acc(deg_hbm)

    return k(x2, gi3d, dst2d, zrow, ones, iden)


def _tc_body(x_ref, a0_ref, a1_ref, d0_ref, d1_ref,
             wst_ref, wn0_ref, wn1_ref, b_ref, o_ref):
    deg = d0_ref[:, 0:1] + d1_ref[:, 0:1]
    recip = 1.0 / jnp.maximum(deg, 1.0)
    dot = functools.partial(jnp.dot, precision=jax.lax.Precision.HIGHEST,
                            preferred_element_type=jnp.float32)
    neigh = dot(a0_ref[...], wn0_ref[...]) + dot(a1_ref[...], wn1_ref[...])
    o_ref[...] = dot(x_ref[...], wst_ref[...]) + neigh * recip + b_ref[...]


def kernel(x, edge_index, W_self, b_self, W_neigh, b_neigh):
    n, d = x.shape
    e = edge_index.shape[1]
    epg = NS * K * G * 2
    e_pad = ((e + epg - 1) // epg) * epg
    n_chunks = e_pad // K
    n_pad = ((n + 1 + 127) // 128) * 128
    rpt = n_pad // NS

    x2 = x.reshape(n * 2, H)
    pad = e_pad - e
    src_p = jnp.concatenate([edge_index[0], jnp.zeros((pad,), jnp.int32)])
    dst_p = jnp.concatenate([edge_index[1], jnp.full((pad,), n, jnp.int32)])
    gi3d = (2 * src_p + jnp.array([[0], [1]], jnp.int32)).reshape(NC, n_chunks, K)
    dst2d = dst_p.reshape(n_chunks, K)
    zrow = jnp.zeros((K, H), jnp.float32)
    ones = jnp.ones((K, H), jnp.float32)
    starts = jnp.array(_zero_chunk_starts(rpt), jnp.int32)
    iden = (jnp.arange(NS, dtype=jnp.int32)[:, None, None] * rpt
            + starts[None, :, None]
            + jnp.arange(K, dtype=jnp.int32)[None, None, :]
            ).reshape(NS, len(_zero_chunk_starts(rpt)), 1, K)

    agg, degp = _sc_aggregate(x2, gi3d, dst2d, zrow, ones, iden,
                              n_pad, n_chunks)

    wst = W_self.T
    wn0 = W_neigh[:, :H].T
    wn1 = W_neigh[:, H:].T
    bias = (b_self + b_neigh).reshape(1, d)

    bn = 256
    nb = (n + bn - 1) // bn
    out = pl.pallas_call(
        _tc_body,
        grid=(nb,),
        in_specs=[
            pl.BlockSpec((bn, d), lambda i: (i, 0)),
            pl.BlockSpec((bn, H), lambda i: (i, 0)),
            pl.BlockSpec((bn, H), lambda i: (i, 0)),
            pl.BlockSpec((bn, H), lambda i: (i, 0)),
            pl.BlockSpec((bn, H), lambda i: (i, 0)),
            pl.BlockSpec((d, d), lambda i: (0, 0)),
            pl.BlockSpec((H, d), lambda i: (0, 0)),
            pl.BlockSpec((H, d), lambda i: (0, 0)),
            pl.BlockSpec((1, d), lambda i: (0, 0)),
        ],
        out_specs=pl.BlockSpec((bn, d), lambda i: (i, 0)),
        out_shape=jax.ShapeDtypeStruct((n, d), jnp.float32),
    )(x, agg[0], agg[1], degp[0], degp[1], wst, wn0, wn1, bias)
    return out


# speedup vs baseline: 3.6181x; 1.0855x over previous
import dataclasses
import functools

import jax
import jax.numpy as jnp
from jax import lax
from jax.experimental import pallas as pl
from jax.experimental.pallas import tpu as pltpu
from jax.experimental.pallas import tpu_sc as plsc

NC = 2
NS = 16
K = 128
H = 128
G = 8


def _zero_chunk_starts(rpt):
    starts = [t * K for t in range((rpt + K - 1) // K)]
    starts[-1] = rpt - K
    return starts


def _sc_aggregate(x2, gi3d, dst2d, zrow, iden, n_pad, n_chunks):
    rpt = n_pad // NS
    cpt = n_chunks // NS
    ngrp = cpt // G
    starts = _zero_chunk_starts(rpt)
    mesh = plsc.VectorSubcoreMesh(core_axis_name="core", subcore_axis_name="subcore")
    cp = pltpu.CompilerParams()
    if "needs_layout_passes" in pltpu.CompilerParams.__dataclass_fields__:
        cp = dataclasses.replace(cp, needs_layout_passes=False)

    @functools.partial(
        pl.kernel,
        compiler_params=cp,
        out_type=[
            jax.ShapeDtypeStruct((NC, n_pad, H), jnp.float32),
            jax.ShapeDtypeStruct((NC, NS, n_pad), jnp.float32),
        ],
        mesh=mesh,
        scratch_types=[
            pltpu.VMEM((G, K), jnp.int32),
            pltpu.VMEM((G, K), jnp.int32),
            pltpu.VMEM((K, H), jnp.float32),
            pltpu.VMEM((K, H), jnp.float32),
            pltpu.VMEM((1, K), jnp.int32),
            pltpu.VMEM((n_pad,), jnp.float32),
            pltpu.VMEM_SHARED((n_pad, H), jnp.float32),
            pltpu.SemaphoreType.DMA,
            pltpu.SemaphoreType.DMA,
            pltpu.SemaphoreType.DMA,
        ],
    )
    def k(x2_hbm, gi_hbm, dst_hbm, zrow_hbm, iden_hbm,
          agg_hbm, deg_hbm,
          sidx, didx, gbufA, gbufB, zidx, degv, acc_sh, gsem, ssemA, ssemB):
        c = lax.axis_index("core")
        s = lax.axis_index("subcore")
        row0 = s * cpt

        def zero_acc():
            pltpu.sync_copy(zrow_hbm, gbufA)
            for ti in range(len(starts)):
                pltpu.sync_copy(iden_hbm.at[s, ti], zidx)
                pltpu.sync_copy(gbufA, acc_sh.at[zidx.at[0]])

        def write_acc(out_hbm):
            for ti, st in enumerate(starts):
                ro = pl.multiple_of(s * rpt + st, 8)
                pltpu.sync_copy(iden_hbm.at[s, ti], zidx)
                pltpu.async_copy(acc_sh.at[zidx.at[0]], gbufA, gsem).wait()
                pltpu.sync_copy(gbufA, out_hbm.at[c, pl.ds(ro, K)])

        zero_acc()

        @pl.loop(0, n_pad, step=16)
        def _(i):
            degv[pl.ds(i, 16)] = jnp.zeros((16,), jnp.float32)

        ones16 = jnp.ones((16,), jnp.float32)
        plsc.subcore_barrier()

        bufs = (gbufA, gbufB)
        ssems = (ssemA, ssemB)

        @pl.loop(0, ngrp)
        def _(g):
            base = pl.multiple_of(row0 + g * G, 8)
            pltpu.sync_copy(gi_hbm.at[c, pl.ds(base, G)], sidx)
            pltpu.sync_copy(dst_hbm.at[pl.ds(base, G)], didx)
            sdesc = [None, None]
            gdesc = [None, None]
            gdesc[0] = pltpu.async_copy(x2_hbm.at[sidx.at[0]], bufs[0], gsem)
            for r in range(G):
                b = r & 1
                nb = 1 - b
                if r + 1 < G:
                    if sdesc[nb] is not None:
                        sdesc[nb].wait()
                    gdesc[nb] = pltpu.async_copy(x2_hbm.at[sidx.at[r + 1]],
                                                 bufs[nb], gsem)
                gdesc[b].wait()
                sdesc[b] = pltpu.async_copy(bufs[b], acc_sh.at[didx.at[r]],
                                            ssems[b], add=True)
                for j in range(0, K, 16):
                    plsc.addupdate_scatter(degv, [didx[r, pl.ds(j, 16)]],
                                           ones16)
            sdesc[0].wait()
            sdesc[1].wait()

        pltpu.sync_copy(degv, deg_hbm.at[c, s])

        plsc.subcore_barrier()
        write_acc(agg_hbm)

    return k(x2, gi3d, dst2d, zrow, iden)


def _tc_body(x_ref, a0_ref, a1_ref, dt_ref,
             wst_ref, wn0_ref, wn1_ref, b_ref, o_ref):
    deg = jnp.sum(dt_ref[...], axis=1, keepdims=True) * 0.5
    recip = 1.0 / jnp.maximum(deg, 1.0)
    dot = functools.partial(jnp.dot, precision=jax.lax.Precision.HIGHEST,
                            preferred_element_type=jnp.float32)
    neigh = dot(a0_ref[...], wn0_ref[...]) + dot(a1_ref[...], wn1_ref[...])
    o_ref[...] = dot(x_ref[...], wst_ref[...]) + neigh * recip + b_ref[...]


def kernel(x, edge_index, W_self, b_self, W_neigh, b_neigh):
    n, d = x.shape
    e = edge_index.shape[1]
    epg = NS * K * G * 2
    e_pad = ((e + epg - 1) // epg) * epg
    n_chunks = e_pad // K
    n_pad = ((n + 1 + 127) // 128) * 128
    rpt = n_pad // NS

    x2 = x.reshape(n * 2, H)
    pad = e_pad - e
    src_p = jnp.concatenate([edge_index[0], jnp.zeros((pad,), jnp.int32)])
    dst_p = jnp.concatenate([edge_index[1], jnp.full((pad,), n, jnp.int32)])
    gi3d = (2 * src_p + jnp.array([[0], [1]], jnp.int32)).reshape(NC, n_chunks, K)
    dst2d = dst_p.reshape(n_chunks, K)
    zrow = jnp.zeros((K, H), jnp.float32)
    starts = jnp.array(_zero_chunk_starts(rpt), jnp.int32)
    iden = (jnp.arange(NS, dtype=jnp.int32)[:, None, None] * rpt
            + starts[None, :, None]
            + jnp.arange(K, dtype=jnp.int32)[None, None, :]
            ).reshape(NS, len(_zero_chunk_starts(rpt)), 1, K)

    agg, degp = _sc_aggregate(x2, gi3d, dst2d, zrow, iden, n_pad, n_chunks)
    degt = degp.reshape(NC * NS, n_pad).T

    wst = W_self.T
    wn0 = W_neigh[:, :H].T
    wn1 = W_neigh[:, H:].T
    bias = (b_self + b_neigh).reshape(1, d)

    bn = 256
    nb = (n + bn - 1) // bn
    out = pl.pallas_call(
        _tc_body,
        grid=(nb,),
        in_specs=[
            pl.BlockSpec((bn, d), lambda i: (i, 0)),
            pl.BlockSpec((bn, H), lambda i: (i, 0)),
            pl.BlockSpec((bn, H), lambda i: (i, 0)),
            pl.BlockSpec((bn, NC * NS), lambda i: (i, 0)),
            pl.BlockSpec((d, d), lambda i: (0, 0)),
            pl.BlockSpec((H, d), lambda i: (0, 0)),
            pl.BlockSpec((H, d), lambda i: (0, 0)),
            pl.BlockSpec((1, d), lambda i: (0, 0)),
        ],
        out_specs=pl.BlockSpec((bn, d), lambda i: (i, 0)),
        out_shape=jax.ShapeDtypeStruct((n, d), jnp.float32),
    )(x, agg[0], agg[1], degt, wst, wn0, wn1, bias)
    return out


# speedup vs baseline: 3.6980x; 1.0221x over previous
import dataclasses
import functools

import jax
import jax.numpy as jnp
from jax import lax
from jax.experimental import pallas as pl
from jax.experimental.pallas import tpu as pltpu
from jax.experimental.pallas import tpu_sc as plsc

NC = 2
NS = 16
K = 128
H = 128
G = 8


def _zero_chunk_starts(rpt):
    starts = [t * K for t in range((rpt + K - 1) // K)]
    starts[-1] = rpt - K
    return starts


def _sc_aggregate(x2, gi3d, dst2d, zrow, iden, n_pad, n_chunks):
    rpt = n_pad // NS
    cpt = n_chunks // NS
    ngrp = cpt // G
    starts = _zero_chunk_starts(rpt)
    mesh = plsc.VectorSubcoreMesh(core_axis_name="core", subcore_axis_name="subcore")
    cp = pltpu.CompilerParams()
    if "needs_layout_passes" in pltpu.CompilerParams.__dataclass_fields__:
        cp = dataclasses.replace(cp, needs_layout_passes=False)

    @functools.partial(
        pl.kernel,
        compiler_params=cp,
        out_type=[
            jax.ShapeDtypeStruct((NC, n_pad, H), jnp.float32),
            jax.ShapeDtypeStruct((NC, NS, n_pad), jnp.float32),
        ],
        mesh=mesh,
        scratch_types=[
            pltpu.VMEM((2, G, K), jnp.int32),
            pltpu.VMEM((2, G, K), jnp.int32),
            pltpu.VMEM((K, H), jnp.float32),
            pltpu.VMEM((K, H), jnp.float32),
            pltpu.VMEM((1, K), jnp.int32),
            pltpu.VMEM((n_pad,), jnp.float32),
            pltpu.VMEM_SHARED((n_pad, H), jnp.float32),
            pltpu.SemaphoreType.DMA,
            pltpu.SemaphoreType.DMA,
            pltpu.SemaphoreType.DMA,
            pltpu.SemaphoreType.DMA,
            pltpu.SemaphoreType.DMA,
        ],
    )
    def k(x2_hbm, gi_hbm, dst_hbm, zrow_hbm, iden_hbm,
          agg_hbm, deg_hbm,
          sidx, didx, gbufA, gbufB, zidx, degv, acc_sh,
          gsem, ssemA, ssemB, isemA, isemB):
        c = lax.axis_index("core")
        s = lax.axis_index("subcore")
        row0 = s * cpt

        def zero_acc():
            pltpu.sync_copy(zrow_hbm, gbufA)
            for ti in range(len(starts)):
                pltpu.sync_copy(iden_hbm.at[s, ti], zidx)
                pltpu.sync_copy(gbufA, acc_sh.at[zidx.at[0]])

        def write_acc(out_hbm):
            for ti, st in enumerate(starts):
                ro = pl.multiple_of(s * rpt + st, 8)
                pltpu.sync_copy(iden_hbm.at[s, ti], zidx)
                pltpu.async_copy(acc_sh.at[zidx.at[0]], gbufA, gsem).wait()
                pltpu.sync_copy(gbufA, out_hbm.at[c, pl.ds(ro, K)])

        zero_acc()

        @pl.loop(0, n_pad, step=16)
        def _(i):
            degv[pl.ds(i, 16)] = jnp.zeros((16,), jnp.float32)

        ones16 = jnp.ones((16,), jnp.float32)
        plsc.subcore_barrier()

        bufs = (gbufA, gbufB)
        ssems = (ssemA, ssemB)

        def issue_idx(slot, grp):
            base = pl.multiple_of(row0 + grp * G, 8)
            return (pltpu.async_copy(gi_hbm.at[c, pl.ds(base, G)],
                                     sidx.at[slot], isemA),
                    pltpu.async_copy(dst_hbm.at[pl.ds(base, G)],
                                     didx.at[slot], isemB))

        idesc = [None, None]
        idesc[0] = issue_idx(0, 0)
        sdesc = [None, None]
        gdesc = [None, None]
        for g in range(ngrp):
            sl = g & 1
            if g == 0:
                for d_ in idesc[0]:
                    d_.wait()
                gdesc[0] = pltpu.async_copy(x2_hbm.at[sidx.at[0, 0]],
                                            bufs[0], gsem)
            for r in range(G):
                b = (g * G + r) & 1
                nb = 1 - b
                nxt = (g, r + 1) if r + 1 < G else (g + 1, 0)
                if nxt[0] < ngrp:
                    if sdesc[nb] is not None:
                        sdesc[nb].wait()
                    if nxt[1] == 0:
                        for d_ in idesc[nxt[0] & 1]:
                            d_.wait()
                    gdesc[nb] = pltpu.async_copy(
                        x2_hbm.at[sidx.at[nxt[0] & 1, nxt[1]]], bufs[nb], gsem)
                if r == 1 and g + 1 < ngrp:
                    idesc[1 - sl] = issue_idx(1 - sl, g + 1)
                gdesc[b].wait()
                sdesc[b] = pltpu.async_copy(bufs[b], acc_sh.at[didx.at[sl, r]],
                                            ssems[b], add=True)

                @pl.loop(0, K, step=16)
                def _(j):
                    plsc.addupdate_scatter(degv, [didx[sl, r, pl.ds(j, 16)]],
                                           ones16)
        sdesc[0].wait()
        sdesc[1].wait()

        pltpu.sync_copy(degv, deg_hbm.at[c, s])

        plsc.subcore_barrier()
        write_acc(agg_hbm)

    return k(x2, gi3d, dst2d, zrow, iden)


def _tc_body(x_ref, a0_ref, a1_ref, dt_ref,
             wst_ref, wn0_ref, wn1_ref, b_ref, o_ref):
    deg = jnp.sum(dt_ref[...], axis=1, keepdims=True) * 0.5
    recip = 1.0 / jnp.maximum(deg, 1.0)
    dot = functools.partial(jnp.dot, precision=jax.lax.Precision.HIGHEST,
                            preferred_element_type=jnp.float32)
    neigh = dot(a0_ref[...], wn0_ref[...]) + dot(a1_ref[...], wn1_ref[...])
    o_ref[...] = dot(x_ref[...], wst_ref[...]) + neigh * recip + b_ref[...]


def kernel(x, edge_index, W_self, b_self, W_neigh, b_neigh):
    n, d = x.shape
    e = edge_index.shape[1]
    epg = NS * K * G * 2
    e_pad = ((e + epg - 1) // epg) * epg
    n_chunks = e_pad // K
    n_pad = ((n + 1 + 127) // 128) * 128
    rpt = n_pad // NS

    x2 = x.reshape(n * 2, H)
    pad = e_pad - e
    src_p = jnp.concatenate([edge_index[0], jnp.zeros((pad,), jnp.int32)])
    dst_p = jnp.concatenate([edge_index[1], jnp.full((pad,), n, jnp.int32)])
    gi3d = (2 * src_p + jnp.array([[0], [1]], jnp.int32)).reshape(NC, n_chunks, K)
    dst2d = dst_p.reshape(n_chunks, K)
    zrow = jnp.zeros((K, H), jnp.float32)
    starts = jnp.array(_zero_chunk_starts(rpt), jnp.int32)
    iden = (jnp.arange(NS, dtype=jnp.int32)[:, None, None] * rpt
            + starts[None, :, None]
            + jnp.arange(K, dtype=jnp.int32)[None, None, :]
            ).reshape(NS, len(_zero_chunk_starts(rpt)), 1, K)

    agg, degp = _sc_aggregate(x2, gi3d, dst2d, zrow, iden, n_pad, n_chunks)
    degt = degp.reshape(NC * NS, n_pad).T

    wst = W_self.T
    wn0 = W_neigh[:, :H].T
    wn1 = W_neigh[:, H:].T
    bias = (b_self + b_neigh).reshape(1, d)

    bn = 256
    nb = (n + bn - 1) // bn
    out = pl.pallas_call(
        _tc_body,
        grid=(nb,),
        in_specs=[
            pl.BlockSpec((bn, d), lambda i: (i, 0)),
            pl.BlockSpec((bn, H), lambda i: (i, 0)),
            pl.BlockSpec((bn, H), lambda i: (i, 0)),
            pl.BlockSpec((bn, NC * NS), lambda i: (i, 0)),
            pl.BlockSpec((d, d), lambda i: (0, 0)),
            pl.BlockSpec((H, d), lambda i: (0, 0)),
            pl.BlockSpec((H, d), lambda i: (0, 0)),
            pl.BlockSpec((1, d), lambda i: (0, 0)),
        ],
        out_specs=pl.BlockSpec((bn, d), lambda i: (i, 0)),
        out_shape=jax.ShapeDtypeStruct((n, d), jnp.float32),
    )(x, agg[0], agg[1], degt, wst, wn0, wn1, bias)
    return out


# speedup vs baseline: 3.8786x; 1.0488x over previous
import dataclasses
import functools

import jax
import jax.numpy as jnp
from jax import lax
from jax.experimental import pallas as pl
from jax.experimental.pallas import tpu as pltpu
from jax.experimental.pallas import tpu_sc as plsc

NC = 2
NS = 16
K = 128
H = 128
G = 8


def _zero_chunk_starts(rpt):
    starts = [t * K for t in range((rpt + K - 1) // K)]
    starts[-1] = rpt - K
    return starts


def _sc_aggregate(x2, gi3d, dst2d, zrow, iden, n_pad, n_chunks):
    rpt = n_pad // NS
    cpt = n_chunks // NS
    ngrp = cpt // G
    starts = _zero_chunk_starts(rpt)
    mesh = plsc.VectorSubcoreMesh(core_axis_name="core", subcore_axis_name="subcore")
    cp = pltpu.CompilerParams()
    if "needs_layout_passes" in pltpu.CompilerParams.__dataclass_fields__:
        cp = dataclasses.replace(cp, needs_layout_passes=False)

    @functools.partial(
        pl.kernel,
        compiler_params=cp,
        out_type=[
            jax.ShapeDtypeStruct((NC, n_pad, H), jnp.float32),
            jax.ShapeDtypeStruct((NC, NS, n_pad), jnp.float32),
        ],
        mesh=mesh,
        scratch_types=[
            pltpu.VMEM((2, G, K), jnp.int32),
            pltpu.VMEM((2, G, K), jnp.int32),
            pltpu.VMEM((K, H), jnp.float32),
            pltpu.VMEM((K, H), jnp.float32),
            pltpu.VMEM((1, K), jnp.int32),
            pltpu.VMEM((n_pad,), jnp.float32),
            pltpu.VMEM_SHARED((n_pad, H), jnp.float32),
            pltpu.SemaphoreType.DMA,
            pltpu.SemaphoreType.DMA,
            pltpu.SemaphoreType.DMA,
            pltpu.SemaphoreType.DMA,
            pltpu.SemaphoreType.DMA,
        ],
    )
    def k(x2_hbm, gi_hbm, dst_hbm, zrow_hbm, iden_hbm,
          agg_hbm, deg_hbm,
          sidx, didx, gbufA, gbufB, zidx, degv, acc_sh,
          gsem, ssemA, ssemB, isemA, isemB):
        c = lax.axis_index("core")
        s = lax.axis_index("subcore")
        row0 = s * cpt

        def zero_acc():
            pltpu.sync_copy(zrow_hbm, gbufA)
            for ti in range(len(starts)):
                pltpu.sync_copy(iden_hbm.at[s, ti], zidx)
                pltpu.sync_copy(gbufA, acc_sh.at[zidx.at[0]])

        def write_acc(out_hbm):
            for ti, st in enumerate(starts):
                ro = pl.multiple_of(s * rpt + st, 8)
                pltpu.sync_copy(iden_hbm.at[s, ti], zidx)
                pltpu.async_copy(acc_sh.at[zidx.at[0]], gbufA, gsem).wait()
                pltpu.sync_copy(gbufA, out_hbm.at[c, pl.ds(ro, K)])

        zero_acc()

        @pl.loop(0, n_pad, step=16)
        def _(i):
            degv[pl.ds(i, 16)] = jnp.zeros((16,), jnp.float32)

        ones16 = jnp.ones((16,), jnp.float32)
        plsc.subcore_barrier()

        bufs = (gbufA, gbufB)
        ssems = (ssemA, ssemB)

        def issue_idx(slot, grp):
            base = pl.multiple_of(row0 + grp * G, 8)
            return (pltpu.async_copy(gi_hbm.at[c, pl.ds(base, G)],
                                     sidx.at[slot], isemA),
                    pltpu.async_copy(dst_hbm.at[pl.ds(base, G)],
                                     didx.at[slot], isemB))

        idesc = [None, None]
        idesc[0] = issue_idx(0, 0)
        sdesc = [None, None]
        gdesc = [None, None]
        for g in range(ngrp):
            sl = g & 1
            if g == 0:
                for d_ in idesc[0]:
                    d_.wait()
                gdesc[0] = pltpu.async_copy(x2_hbm.at[sidx.at[0, 0]],
                                            bufs[0], gsem)
            for r in range(G):
                b = (g * G + r) & 1
                nb = 1 - b
                nxt = (g, r + 1) if r + 1 < G else (g + 1, 0)
                if nxt[0] < ngrp:
                    if sdesc[nb] is not None:
                        sdesc[nb].wait()
                    if nxt[1] == 0:
                        for d_ in idesc[nxt[0] & 1]:
                            d_.wait()
                    gdesc[nb] = pltpu.async_copy(
                        x2_hbm.at[sidx.at[nxt[0] & 1, nxt[1]]], bufs[nb], gsem)
                if r == 1 and g + 1 < ngrp:
                    idesc[1 - sl] = issue_idx(1 - sl, g + 1)
                gdesc[b].wait()
                sdesc[b] = pltpu.async_copy(bufs[b], acc_sh.at[didx.at[sl, r]],
                                            ssems[b], add=True)

                @pl.loop(0, K, step=16)
                def _(j):
                    plsc.addupdate_scatter(degv, [didx[sl, r, pl.ds(j, 16)]],
                                           ones16)
        sdesc[0].wait()
        sdesc[1].wait()

        pltpu.sync_copy(degv, deg_hbm.at[c, s])

        plsc.subcore_barrier()
        write_acc(agg_hbm)

    return k(x2, gi3d, dst2d, zrow, iden)


def _tc_body(x_ref, a0_ref, a1_ref, dt_ref,
             wst_ref, wn0_ref, wn1_ref, b_ref, o_ref):
    deg = jnp.sum(dt_ref[...], axis=1, keepdims=True) * 0.5
    recip = 1.0 / jnp.maximum(deg, 1.0)
    dot = functools.partial(jnp.dot, preferred_element_type=jnp.float32)
    neigh = dot(a0_ref[...], wn0_ref[...]) + dot(a1_ref[...], wn1_ref[...])
    o_ref[...] = dot(x_ref[...], wst_ref[...]) + neigh * recip + b_ref[...]


def kernel(x, edge_index, W_self, b_self, W_neigh, b_neigh):
    n, d = x.shape
    e = edge_index.shape[1]
    epg = NS * K * G * 2
    e_pad = ((e + epg - 1) // epg) * epg
    n_chunks = e_pad // K
    n_pad = ((n + 1 + 127) // 128) * 128
    rpt = n_pad // NS

    x2 = x.reshape(n * 2, H)
    pad = e_pad - e
    src_p = jnp.concatenate([edge_index[0], jnp.zeros((pad,), jnp.int32)])
    dst_p = jnp.concatenate([edge_index[1], jnp.full((pad,), n, jnp.int32)])
    gi3d = (2 * src_p + jnp.array([[0], [1]], jnp.int32)).reshape(NC, n_chunks, K)
    dst2d = dst_p.reshape(n_chunks, K)
    zrow = jnp.zeros((K, H), jnp.float32)
    starts = jnp.array(_zero_chunk_starts(rpt), jnp.int32)
    iden = (jnp.arange(NS, dtype=jnp.int32)[:, None, None] * rpt
            + starts[None, :, None]
            + jnp.arange(K, dtype=jnp.int32)[None, None, :]
            ).reshape(NS, len(_zero_chunk_starts(rpt)), 1, K)

    agg, degp = _sc_aggregate(x2, gi3d, dst2d, zrow, iden, n_pad, n_chunks)
    degt = degp.reshape(NC * NS, n_pad).T

    wst = W_self.T
    wn0 = W_neigh[:, :H].T
    wn1 = W_neigh[:, H:].T
    bias = (b_self + b_neigh).reshape(1, d)

    bn = 256
    nb = (n + bn - 1) // bn
    out = pl.pallas_call(
        _tc_body,
        grid=(nb,),
        in_specs=[
            pl.BlockSpec((bn, d), lambda i: (i, 0)),
            pl.BlockSpec((bn, H), lambda i: (i, 0)),
            pl.BlockSpec((bn, H), lambda i: (i, 0)),
            pl.BlockSpec((bn, NC * NS), lambda i: (i, 0)),
            pl.BlockSpec((d, d), lambda i: (0, 0)),
            pl.BlockSpec((H, d), lambda i: (0, 0)),
            pl.BlockSpec((H, d), lambda i: (0, 0)),
            pl.BlockSpec((1, d), lambda i: (0, 0)),
        ],
        out_specs=pl.BlockSpec((bn, d), lambda i: (i, 0)),
        out_shape=jax.ShapeDtypeStruct((n, d), jnp.float32),
    )(x, agg[0], agg[1], degt, wst, wn0, wn1, bias)
    return out


# speedup vs baseline: 3.8992x; 1.0053x over previous
import dataclasses
import functools

import jax
import jax.numpy as jnp
from jax import lax
from jax.experimental import pallas as pl
from jax.experimental.pallas import tpu as pltpu
from jax.experimental.pallas import tpu_sc as plsc

NC = 2
NS = 16
K = 128
H = 128
G = 8


def _zero_chunk_starts(rpt):
    starts = [t * K for t in range((rpt + K - 1) // K)]
    starts[-1] = rpt - K
    return starts


def _sc_aggregate(x2, gi3d, dst2d, zrow, iden, n_pad, n_chunks):
    rpt = n_pad // NS
    cpt = n_chunks // NS
    ngrp = cpt // G
    starts = _zero_chunk_starts(rpt)
    mesh = plsc.VectorSubcoreMesh(core_axis_name="core", subcore_axis_name="subcore")
    cp = pltpu.CompilerParams()
    if "needs_layout_passes" in pltpu.CompilerParams.__dataclass_fields__:
        cp = dataclasses.replace(cp, needs_layout_passes=False)

    @functools.partial(
        pl.kernel,
        compiler_params=cp,
        out_type=[
            jax.ShapeDtypeStruct((NC, n_pad, H), jnp.float32),
            jax.ShapeDtypeStruct((NC, NS, n_pad), jnp.float32),
        ],
        mesh=mesh,
        scratch_types=[
            pltpu.VMEM((2, G, K), jnp.int32),
            pltpu.VMEM((2, G, K), jnp.int32),
            pltpu.VMEM((K, H), jnp.float32),
            pltpu.VMEM((K, H), jnp.float32),
            pltpu.VMEM((1, K), jnp.int32),
            pltpu.VMEM((n_pad,), jnp.float32),
            pltpu.VMEM_SHARED((n_pad, H), jnp.float32),
            pltpu.SemaphoreType.DMA,
            pltpu.SemaphoreType.DMA,
            pltpu.SemaphoreType.DMA,
            pltpu.SemaphoreType.DMA,
            pltpu.SemaphoreType.DMA,
        ],
    )
    def k(x2_hbm, gi_hbm, dst_hbm, zrow_hbm, iden_hbm,
          agg_hbm, deg_hbm,
          sidx, didx, gbufA, gbufB, zidx, degv, acc_sh,
          gsem, ssemA, ssemB, isemA, isemB):
        c = lax.axis_index("core")
        s = lax.axis_index("subcore")
        row0 = s * cpt

        def zero_acc():
            pltpu.sync_copy(zrow_hbm, gbufA)
            for ti in range(len(starts)):
                pltpu.sync_copy(iden_hbm.at[s, ti], zidx)
                pltpu.sync_copy(gbufA, acc_sh.at[zidx.at[0]])

        def write_acc(out_hbm):
            for ti, st in enumerate(starts):
                ro = pl.multiple_of(s * rpt + st, 8)
                pltpu.sync_copy(iden_hbm.at[s, ti], zidx)
                pltpu.async_copy(acc_sh.at[zidx.at[0]], gbufA, gsem).wait()
                pltpu.sync_copy(gbufA, out_hbm.at[c, pl.ds(ro, K)])

        zero_acc()

        @pl.loop(0, n_pad, step=16)
        def _(i):
            degv[pl.ds(i, 16)] = jnp.zeros((16,), jnp.float32)

        ones16 = jnp.ones((16,), jnp.float32)
        plsc.subcore_barrier()

        bufs = (gbufA, gbufB)
        ssems = (ssemA, ssemB)

        def issue_idx(slot, grp):
            base = pl.multiple_of(row0 + grp * G, 8)
            return (pltpu.async_copy(gi_hbm.at[c, pl.ds(base, G)],
                                     sidx.at[slot], isemA),
                    pltpu.async_copy(dst_hbm.at[pl.ds(base, G)],
                                     didx.at[slot], isemB))

        idesc = [None, None]
        idesc[0] = issue_idx(0, 0)
        sdesc = [None, None]
        gdesc = [None, None]
        for g in range(ngrp):
            sl = g & 1
            if g == 0:
                for d_ in idesc[0]:
                    d_.wait()
                gdesc[0] = pltpu.async_copy(x2_hbm.at[sidx.at[0, 0]],
                                            bufs[0], gsem)
            for r in range(G):
                b = (g * G + r) & 1
                nb = 1 - b
                nxt = (g, r + 1) if r + 1 < G else (g + 1, 0)
                if nxt[0] < ngrp:
                    if sdesc[nb] is not None:
                        sdesc[nb].wait()
                    if nxt[1] == 0:
                        for d_ in idesc[nxt[0] & 1]:
                            d_.wait()
                    gdesc[nb] = pltpu.async_copy(
                        x2_hbm.at[sidx.at[nxt[0] & 1, nxt[1]]], bufs[nb], gsem)
                if r == 1 and g + 1 < ngrp:
                    idesc[1 - sl] = issue_idx(1 - sl, g + 1)
                gdesc[b].wait()
                sdesc[b] = pltpu.async_copy(bufs[b], acc_sh.at[didx.at[sl, r]],
                                            ssems[b], add=True)

                @pl.loop(0, K, step=16)
                def _(j):
                    plsc.addupdate_scatter(degv, [didx[sl, r, pl.ds(j, 16)]],
                                           ones16)
        sdesc[0].wait()
        sdesc[1].wait()

        pltpu.sync_copy(degv, deg_hbm.at[c, s])

        plsc.subcore_barrier()
        write_acc(agg_hbm)

    return k(x2, gi3d, dst2d, zrow, iden)


def _tc_self_body(x_ref, wst_ref, b_ref, o_ref):
    dot = functools.partial(jnp.dot, preferred_element_type=jnp.float32)
    o_ref[...] = dot(x_ref[...], wst_ref[...]) + b_ref[...]


def _tc_combine_body(s_ref, a0_ref, a1_ref, dt_ref, wn0_ref, wn1_ref, o_ref):
    deg = jnp.sum(dt_ref[...], axis=1, keepdims=True) * 0.5
    recip = 1.0 / jnp.maximum(deg, 1.0)
    dot = functools.partial(jnp.dot, preferred_element_type=jnp.float32)
    neigh = dot(a0_ref[...], wn0_ref[...]) + dot(a1_ref[...], wn1_ref[...])
    o_ref[...] = s_ref[...] + neigh * recip


def kernel(x, edge_index, W_self, b_self, W_neigh, b_neigh):
    n, d = x.shape
    e = edge_index.shape[1]
    epg = NS * K * G * 2
    e_pad = ((e + epg - 1) // epg) * epg
    n_chunks = e_pad // K
    n_pad = ((n + 1 + 127) // 128) * 128
    rpt = n_pad // NS

    x2 = x.reshape(n * 2, H)
    pad = e_pad - e
    src_p = jnp.concatenate([edge_index[0], jnp.zeros((pad,), jnp.int32)])
    dst_p = jnp.concatenate([edge_index[1], jnp.full((pad,), n, jnp.int32)])
    gi3d = (2 * src_p + jnp.array([[0], [1]], jnp.int32)).reshape(NC, n_chunks, K)
    dst2d = dst_p.reshape(n_chunks, K)
    zrow = jnp.zeros((K, H), jnp.float32)
    starts = jnp.array(_zero_chunk_starts(rpt), jnp.int32)
    iden = (jnp.arange(NS, dtype=jnp.int32)[:, None, None] * rpt
            + starts[None, :, None]
            + jnp.arange(K, dtype=jnp.int32)[None, None, :]
            ).reshape(NS, len(_zero_chunk_starts(rpt)), 1, K)

    wst = W_self.T
    wn0 = W_neigh[:, :H].T
    wn1 = W_neigh[:, H:].T
    bias = (b_self + b_neigh).reshape(1, d)

    bn = 256
    nb = (n + bn - 1) // bn

    self_t = pl.pallas_call(
        _tc_self_body,
        grid=(nb,),
        in_specs=[
            pl.BlockSpec((bn, d), lambda i: (i, 0)),
            pl.BlockSpec((d, d), lambda i: (0, 0)),
            pl.BlockSpec((1, d), lambda i: (0, 0)),
        ],
        out_specs=pl.BlockSpec((bn, d), lambda i: (i, 0)),
        out_shape=jax.ShapeDtypeStruct((n, d), jnp.float32),
    )(x, wst, bias)

    agg, degp = _sc_aggregate(x2, gi3d, dst2d, zrow, iden, n_pad, n_chunks)
    degt = degp.reshape(NC * NS, n_pad).T

    out = pl.pallas_call(
        _tc_combine_body,
        grid=(nb,),
        in_specs=[
            pl.BlockSpec((bn, d), lambda i: (i, 0)),
            pl.BlockSpec((bn, H), lambda i: (i, 0)),
            pl.BlockSpec((bn, H), lambda i: (i, 0)),
            pl.BlockSpec((bn, NC * NS), lambda i: (i, 0)),
            pl.BlockSpec((H, d), lambda i: (0, 0)),
            pl.BlockSpec((H, d), lambda i: (0, 0)),
        ],
        out_specs=pl.BlockSpec((bn, d), lambda i: (i, 0)),
        out_shape=jax.ShapeDtypeStruct((n, d), jnp.float32),
    )(self_t, agg[0], agg[1], degt, wn0, wn1)
    return out
